# Initial kernel scaffold; baseline (speedup 1.0000x reference)
import functools

import jax
import jax.numpy as jnp
from jax import lax
from jax.experimental import pallas as pl
from jax.experimental.pallas import tpu as pltpu
from jax.experimental.pallas import tpu_sc as plsc

NW = 32
TOTAL_IN = 32 * 2 * 480000
TOTAL_OUT = TOTAL_IN * 5 // 4
IN_PER_W = TOTAL_IN // NW
OUT_PER_W = TOTAL_OUT // NW
OUT_CHUNK = 30_000
IN_CHUNK = OUT_CHUNK * 4 // 5
N_CHUNKS = OUT_PER_W // OUT_CHUNK
SUPER = OUT_CHUNK // 80


def _resample_flat(xf):
    mesh = plsc.VectorSubcoreMesh(core_axis_name="c", subcore_axis_name="s")

    @functools.partial(
        pl.kernel,
        out_type=jax.ShapeDtypeStruct((TOTAL_OUT,), jnp.float32),
        mesh=mesh,
        scratch_types=[
            pltpu.VMEM((IN_CHUNK,), jnp.float32),
            pltpu.VMEM((OUT_CHUNK,), jnp.float32),
        ],
    )
    def k(x_hbm, out_hbm, in_v, out_v):
        wid = lax.axis_index("s") * 2 + lax.axis_index("c")
        in_base = wid * IN_PER_W
        out_base = wid * OUT_PER_W

        lane = lax.iota(jnp.int32, 16)
        bases = [lax.div(4 * (16 * i + lane) + 2, 5) for i in range(5)]

        def chunk_body(c, carry):
            pltpu.sync_copy(
                x_hbm.at[pl.ds(in_base + c * IN_CHUNK, IN_CHUNK)], in_v
            )

            def step(s, carry2):
                off = 80 * s
                shift = 64 * s
                for i in range(5):
                    v = plsc.load_gather(in_v, [bases[i] + shift])
                    out_v[pl.ds(off + 16 * i, 16)] = v
                return carry2

            lax.fori_loop(0, SUPER, step, 0, unroll=2)

            pltpu.sync_copy(
                out_v, out_hbm.at[pl.ds(out_base + c * OUT_CHUNK, OUT_CHUNK)]
            )
            return carry

        lax.fori_loop(0, N_CHUNKS, chunk_body, 0)

    return k(xf)


def kernel(x):
    b, ch, _ = x.shape
    out = _resample_flat(x.reshape(-1))
    return out.reshape(b, ch, TOTAL_OUT // (b * ch))


# speedup vs baseline: 17.8458x; 17.8458x over previous
import functools

import jax
import jax.numpy as jnp
from jax import lax
from jax.experimental import pallas as pl
from jax.experimental.pallas import tpu as pltpu
from jax.experimental.pallas import tpu_sc as plsc

NW = 32
TOTAL_IN = 32 * 2 * 480000
TOTAL_OUT = TOTAL_IN * 5 // 4
IN_PER_W = TOTAL_IN // NW
OUT_PER_W = TOTAL_OUT // NW
OUT_CHUNK = 30_000
IN_CHUNK = OUT_CHUNK * 4 // 5
N_CHUNKS = OUT_PER_W // OUT_CHUNK
SUPER = OUT_CHUNK // 80


def _resample_flat(xf):
    mesh = plsc.VectorSubcoreMesh(core_axis_name="c", subcore_axis_name="s")

    @functools.partial(
        pl.kernel,
        out_type=jax.ShapeDtypeStruct((TOTAL_OUT,), jnp.float32),
        mesh=mesh,
        scratch_types=[
            pltpu.VMEM((IN_CHUNK,), jnp.float32),
            pltpu.VMEM((OUT_CHUNK,), jnp.float32),
        ],
        compiler_params=pltpu.CompilerParams(needs_layout_passes=False),
    )
    def k(x_hbm, out_hbm, in_v, out_v):
        wid = lax.axis_index("s") * 2 + lax.axis_index("c")
        in_base = wid * IN_PER_W
        out_base = wid * OUT_PER_W

        lane = lax.iota(jnp.int32, 16)
        bases = [lax.div(4 * (16 * i + lane) + 2, 5) for i in range(5)]

        def chunk_body(c, carry):
            pltpu.sync_copy(
                x_hbm.at[pl.ds(in_base + c * IN_CHUNK, IN_CHUNK)], in_v
            )

            def step(s, carry2):
                off = 80 * s
                shift = 64 * s
                for i in range(5):
                    v = plsc.load_gather(in_v, [bases[i] + shift])
                    out_v[pl.ds(off + 16 * i, 16)] = v
                return carry2

            lax.fori_loop(0, SUPER, step, 0, unroll=2)

            pltpu.sync_copy(
                out_v, out_hbm.at[pl.ds(out_base + c * OUT_CHUNK, OUT_CHUNK)]
            )
            return carry

        lax.fori_loop(0, N_CHUNKS, chunk_body, 0)

    return k(xf)


def kernel(x):
    b, ch, _ = x.shape
    out = _resample_flat(x.reshape(-1))
    return out.reshape(b, ch, TOTAL_OUT // (b * ch))


# speedup vs baseline: 19.3178x; 1.0825x over previous
import functools

import jax
import jax.numpy as jnp
from jax import lax
from jax.experimental import pallas as pl
from jax.experimental.pallas import tpu as pltpu
from jax.experimental.pallas import tpu_sc as plsc

NW = 32
TOTAL_IN = 32 * 2 * 480000
TOTAL_OUT = TOTAL_IN * 5 // 4
IN_PER_W = TOTAL_IN // NW
OUT_PER_W = TOTAL_OUT // NW
OUT_CHUNK = 30_000
IN_CHUNK = OUT_CHUNK * 4 // 5
N_CHUNKS = OUT_PER_W // OUT_CHUNK
SUPER = OUT_CHUNK // 80


def _resample_flat(xf):
    mesh = plsc.VectorSubcoreMesh(core_axis_name="c", subcore_axis_name="s")

    @functools.partial(
        pl.kernel,
        out_type=jax.ShapeDtypeStruct((TOTAL_OUT,), jnp.float32),
        mesh=mesh,
        scratch_types=[
            pltpu.VMEM((IN_CHUNK,), jnp.float32),
            pltpu.VMEM((IN_CHUNK,), jnp.float32),
            pltpu.VMEM((OUT_CHUNK,), jnp.float32),
            pltpu.VMEM((OUT_CHUNK,), jnp.float32),
            pltpu.SemaphoreType.DMA((2,)),
            pltpu.SemaphoreType.DMA((2,)),
        ],
        compiler_params=pltpu.CompilerParams(needs_layout_passes=False),
    )
    def k(x_hbm, out_hbm, in_v0, in_v1, out_v0, out_v1, in_sem, out_sem):
        in_bufs = (in_v0, in_v1)
        out_bufs = (out_v0, out_v1)
        wid = lax.axis_index("s") * 2 + lax.axis_index("c")
        in_base = wid * IN_PER_W
        out_base = wid * OUT_PER_W

        lane = lax.iota(jnp.int32, 16)
        bases = [lax.div(4 * (16 * i + lane) + 2, 5) for i in range(5)]

        def in_copy(c, b):
            return pltpu.make_async_copy(
                x_hbm.at[pl.ds(in_base + c * IN_CHUNK, IN_CHUNK)],
                in_bufs[b],
                in_sem.at[b],
            )

        def out_copy(c, b):
            return pltpu.make_async_copy(
                out_bufs[b],
                out_hbm.at[pl.ds(out_base + c * OUT_CHUNK, OUT_CHUNK)],
                out_sem.at[b],
            )

        in_copy(0, 0).start()
        in_copy(1, 1).start()

        def outer(g, carry):
            for b in range(2):
                c = 2 * g + b
                in_copy(c, b).wait()

                @pl.when(g >= 1)
                def _():
                    out_copy(c - 2, b).wait()

                @plsc.parallel_loop(0, SUPER, unroll=4)
                def step(s):
                    off = 80 * s
                    shift = 64 * s
                    for i in range(5):
                        v = plsc.load_gather(in_bufs[b], [bases[i] + shift])
                        out_bufs[b][pl.ds(off + 16 * i, 16)] = v

                out_copy(c, b).start()

                @pl.when(g + 1 < N_CHUNKS // 2)
                def _():
                    in_copy(c + 2, b).start()

            return carry

        lax.fori_loop(0, N_CHUNKS // 2, outer, 0)
        out_copy(N_CHUNKS - 2, 0).wait()
        out_copy(N_CHUNKS - 1, 1).wait()

    return k(xf)


def kernel(x):
    b, ch, _ = x.shape
    out = _resample_flat(x.reshape(-1))
    return out.reshape(b, ch, TOTAL_OUT // (b * ch))


# speedup vs baseline: 19.3238x; 1.0003x over previous
import functools

import jax
import jax.numpy as jnp
from jax import lax
from jax.experimental import pallas as pl
from jax.experimental.pallas import tpu as pltpu
from jax.experimental.pallas import tpu_sc as plsc

NW = 32
TOTAL_IN = 32 * 2 * 480000
TOTAL_OUT = TOTAL_IN * 5 // 4
IN_PER_W = TOTAL_IN // NW
OUT_PER_W = TOTAL_OUT // NW
OUT_CHUNK = 30_000
IN_CHUNK = OUT_CHUNK * 4 // 5
N_CHUNKS = OUT_PER_W // OUT_CHUNK
SUPER = OUT_CHUNK // 80


def _resample_flat(xf):
    mesh = plsc.VectorSubcoreMesh(core_axis_name="c", subcore_axis_name="s")

    @functools.partial(
        pl.kernel,
        out_type=jax.ShapeDtypeStruct((TOTAL_OUT,), jnp.float32),
        mesh=mesh,
        scratch_types=[
            pltpu.VMEM((IN_CHUNK,), jnp.float32),
            pltpu.VMEM((IN_CHUNK,), jnp.float32),
            pltpu.VMEM((OUT_CHUNK,), jnp.float32),
            pltpu.VMEM((OUT_CHUNK,), jnp.float32),
            pltpu.SemaphoreType.DMA((2,)),
            pltpu.SemaphoreType.DMA((2,)),
        ],
        compiler_params=pltpu.CompilerParams(
            needs_layout_passes=False,
            skip_device_barrier=True,
            disable_bounds_checks=True,
        ),
    )
    def k(x_hbm, out_hbm, in_v0, in_v1, out_v0, out_v1, in_sem, out_sem):
        in_bufs = (in_v0, in_v1)
        out_bufs = (out_v0, out_v1)
        wid = lax.axis_index("s") * 2 + lax.axis_index("c")
        in_base = wid * IN_PER_W
        out_base = wid * OUT_PER_W

        lane = lax.iota(jnp.int32, 16)
        bases = [lax.div(4 * (16 * i + lane) + 2, 5) for i in range(5)]

        def in_copy(c, b):
            return pltpu.make_async_copy(
                x_hbm.at[pl.ds(in_base + c * IN_CHUNK, IN_CHUNK)],
                in_bufs[b],
                in_sem.at[b],
            )

        def out_copy(c, b):
            return pltpu.make_async_copy(
                out_bufs[b],
                out_hbm.at[pl.ds(out_base + c * OUT_CHUNK, OUT_CHUNK)],
                out_sem.at[b],
            )

        in_copy(0, 0).start()
        in_copy(1, 1).start()

        def outer(g, carry):
            for b in range(2):
                c = 2 * g + b
                in_copy(c, b).wait()

                @pl.when(g >= 1)
                def _():
                    out_copy(c - 2, b).wait()

                @plsc.parallel_loop(0, SUPER, unroll=4)
                def step(s):
                    off = 80 * s
                    shift = 64 * s
                    for i in range(5):
                        v = plsc.load_gather(in_bufs[b], [bases[i] + shift])
                        out_bufs[b][pl.ds(off + 16 * i, 16)] = v

                out_copy(c, b).start()

                @pl.when(g + 1 < N_CHUNKS // 2)
                def _():
                    in_copy(c + 2, b).start()

            return carry

        lax.fori_loop(0, N_CHUNKS // 2, outer, 0)
        out_copy(N_CHUNKS - 2, 0).wait()
        out_copy(N_CHUNKS - 1, 1).wait()

    return k(xf)


def kernel(x):
    b, ch, _ = x.shape
    out = _resample_flat(x.reshape(-1))
    return out.reshape(b, ch, TOTAL_OUT // (b * ch))
